# baseline (speedup 1.0000x reference)
import functools

import jax
import jax.numpy as jnp
from jax import lax
from jax.experimental import pallas as pl
from jax.experimental.pallas import tpu as pltpu
from jax.experimental.pallas import tpu_sc as plsc

VOCAB = 1000000
EMBED = 64
B = 4096
L = 200

NC = 2
NS = 16
NW = NC * NS

TOTAL = B * L

VBLK = 128
NVBLK_PAD = 7840
KA = NVBLK_PAD // NW
NBA = 5
GA = KA // NBA
VOCAB_PAD = 1000064
V0_LAST = VOCAB_PAD - VBLK

CHUNK = 128
NCHUNK = TOTAL // CHUNK
KB = NCHUNK // NW
NBG = 3
GB = 67


def _iota16():
  return lax.iota(jnp.int32, 16)


def _make_format_kernel():
  mesh = plsc.VectorSubcoreMesh(
      core_axis_name="c", subcore_axis_name="s", num_cores=NC, num_subcores=NS
  )

  @functools.partial(
      pl.kernel,
      mesh=mesh,
      compiler_params=pltpu.CompilerParams(
          use_tc_tiling_on_sc=True, needs_layout_passes=False,
          disable_bounds_checks=True),
      out_type=jax.ShapeDtypeStruct((VOCAB_PAD // 2, 2 * EMBED), jnp.float32),
      scratch_types=[
          pltpu.VMEM((NBA, EMBED, VBLK), jnp.float32),
          pltpu.VMEM((NBA, VBLK // 2, 2 * EMBED), jnp.float32),
          pltpu.SemaphoreType.DMA((NBA,)),
          pltpu.SemaphoreType.DMA((NBA,)),
      ],
  )
  def k(tt_hbm, out_hbm, n_v, f_v, gsem, fsem):
    wid = lax.axis_index("s") * NC + lax.axis_index("c")

    def v0_of(kk):
      return pl.multiple_of(
          jnp.minimum((wid + NW * kk) * VBLK, V0_LAST), VBLK)

    def rdesc(kk, i):
      return pltpu.make_async_copy(
          tt_hbm.at[:, pl.ds(v0_of(kk), VBLK)], n_v.at[i], gsem.at[i])

    def wdesc(kk, i):
      r0 = pl.multiple_of(v0_of(kk) // 2, VBLK // 2)
      return pltpu.make_async_copy(
          f_v.at[i], out_hbm.at[pl.ds(r0, VBLK // 2), :], fsem.at[i])

    for i in range(NBA):
      rdesc(i, i).start()

    def group(g, carry):
      for i in range(NBA):
        kk = g * NBA + i
        rdesc(kk, i).wait()

        @pl.when(g > 0)
        def _():
          wdesc(kk - NBA, i).wait()

        @plsc.parallel_loop(0, VBLK, step=16, unroll=2)
        def _(vv0):
          for e0 in range(0, EMBED, 16):
            e_idx = _iota16() + e0
            for d in range(16):
              v_idx = vv0 + jnp.bitwise_and(_iota16() + d, 15)
              vals = plsc.load_gather(n_v.at[i], [e_idx, v_idx])
              r_idx = lax.shift_right_logical(v_idx, 1)
              c_idx = jnp.bitwise_and(v_idx, 1) * EMBED + e_idx
              plsc.store_scatter(f_v.at[i], [r_idx, c_idx], vals)
        wdesc(kk, i).start()

        @pl.when(g < GA - 1)
        def _():
          rdesc(kk + NBA, i).start()
      return carry

    lax.fori_loop(0, GA, group, 0)
    for i in range(NBA):
      wdesc((GA - 1) * NBA + i, i).wait()

  return k


def _make_gather_kernel():
  mesh = plsc.VectorSubcoreMesh(
      core_axis_name="c", subcore_axis_name="s", num_cores=NC, num_subcores=NS
  )

  @functools.partial(
      pl.kernel,
      mesh=mesh,
      compiler_params=pltpu.CompilerParams(use_tc_tiling_on_sc=True, needs_layout_passes=False),
      out_type=jax.ShapeDtypeStruct((L, EMBED, B), jnp.float32),
      scratch_types=[
          pltpu.VMEM((KB * CHUNK,), jnp.int32),
          pltpu.VMEM((KB * CHUNK,), jnp.int32),
          pltpu.VMEM((NBG, CHUNK, 2 * EMBED), jnp.float32),
          pltpu.VMEM((NBG, EMBED, CHUNK), jnp.float32),
          pltpu.SemaphoreType.DMA,
          pltpu.SemaphoreType.DMA((NBG,)),
          pltpu.SemaphoreType.DMA((NBG,)),
      ],
  )
  def k(table_hbm, idx_hbm, out_hbm, idx_v, idx2_v, g_v, s_v, isem, gsem,
        ssem):
    wid = lax.axis_index("s") * NC + lax.axis_index("c")
    pltpu.async_copy(
        idx_hbm.at[pl.ds(wid * KB * CHUNK, KB * CHUNK)], idx_v, isem).wait()

    @plsc.parallel_loop(0, KB * CHUNK, step=16, unroll=4)
    def _(o):
      idx2_v[pl.ds(o, 16)] = lax.shift_right_logical(idx_v[pl.ds(o, 16)], 1)

    def jloc(kk):
      return jnp.minimum(kk, KB - 1)

    def gdesc(kk, i):
      return pltpu.make_async_copy(
          table_hbm.at[idx2_v.at[pl.ds(jloc(kk) * CHUNK, CHUNK)]],
          g_v.at[i], gsem.at[i])

    def sdesc(kk, i):
      c = wid * KB + jloc(kk)
      ll = c // (B // CHUNK)
      b0 = pl.multiple_of((c % (B // CHUNK)) * CHUNK, CHUNK)
      return pltpu.make_async_copy(
          s_v.at[i], out_hbm.at[ll, :, pl.ds(b0, CHUNK)], ssem.at[i])

    for i in range(NBG):
      gdesc(i, i).start()

    def group(g, carry):
      for i in range(NBG):
        kk = g * NBG + i
        gdesc(kk, i).wait()

        @pl.when(g > 0)
        def _():
          sdesc(kk - NBG, i).wait()

        base = jloc(kk) * CHUNK

        @plsc.parallel_loop(0, CHUNK, step=16, unroll=2)
        def _(t0):
          t_idx = _iota16() + t0
          par = jnp.bitwise_and(idx_v[pl.ds(base + t0, 16)], 1) * EMBED
          for e0 in range(0, EMBED, 16):
            for d in range(16):
              e_idx = e0 + jnp.bitwise_and(_iota16() + d, 15)
              vals = plsc.load_gather(g_v.at[i], [t_idx, par + e_idx])
              plsc.store_scatter(s_v.at[i], [e_idx, t_idx], vals)
        sdesc(kk, i).start()

        @pl.when(g < GB - 1)
        def _():
          gdesc(kk + NBG, i).start()
      return carry

    lax.fori_loop(0, GB, group, 0)
    for i in range(NBG):
      sdesc((GB - 1) * NBG + i, i).wait()

  return k


_format_call = _make_format_kernel()
_gather_call = _make_gather_kernel()


def kernel(token_ids, embeddings):
  table_t = embeddings.T
  idx_flat = token_ids.T.reshape(TOTAL).astype(jnp.int32)
  table2 = _format_call(table_t)
  out_t = _gather_call(table2, idx_flat)
  return out_t.transpose(2, 0, 1)


# speedup vs baseline: 1.7870x; 1.7870x over previous
import functools

import jax
import jax.numpy as jnp
from jax import lax
from jax.experimental import pallas as pl
from jax.experimental.pallas import tpu as pltpu
from jax.experimental.pallas import tpu_sc as plsc

VOCAB = 1000000
EMBED = 64
B = 4096
L = 200

NC = 2
NS = 16
NW = NC * NS

TOTAL = B * L

VBLK = 128
NVBLK_PAD = 7840
KA = NVBLK_PAD // NW
NBA = 5
GA = KA // NBA
VOCAB_PAD = 1000064
V0_LAST = VOCAB_PAD - VBLK

CHUNK = 128
NCHUNK = TOTAL // CHUNK
KB = NCHUNK // NW
NBG = 3
GB = 67


def _iota16():
  return lax.iota(jnp.int32, 16)


def _make_format_kernel():
  mesh = plsc.VectorSubcoreMesh(
      core_axis_name="c", subcore_axis_name="s", num_cores=NC, num_subcores=NS
  )

  @functools.partial(
      pl.kernel,
      mesh=mesh,
      compiler_params=pltpu.CompilerParams(
          use_tc_tiling_on_sc=True, needs_layout_passes=False,
          disable_bounds_checks=True),
      out_type=jax.ShapeDtypeStruct((VOCAB_PAD // 2, 2 * EMBED), jnp.float32),
      scratch_types=[
          pltpu.VMEM((NBA, EMBED, VBLK), jnp.float32),
          pltpu.VMEM((NBA, VBLK // 2, 2 * EMBED), jnp.float32),
          pltpu.SemaphoreType.DMA((NBA,)),
          pltpu.SemaphoreType.DMA((NBA,)),
      ],
  )
  def k(tt_hbm, out_hbm, n_v, f_v, gsem, fsem):
    wid = lax.axis_index("s") * NC + lax.axis_index("c")

    def v0_of(kk):
      return pl.multiple_of(
          jnp.minimum((wid + NW * kk) * VBLK, V0_LAST), VBLK)

    def rdesc(kk, i):
      return pltpu.make_async_copy(
          tt_hbm.at[:, pl.ds(v0_of(kk), VBLK)], n_v.at[i], gsem.at[i])

    def wdesc(kk, i):
      r0 = pl.multiple_of(v0_of(kk) // 2, VBLK // 2)
      return pltpu.make_async_copy(
          f_v.at[i], out_hbm.at[pl.ds(r0, VBLK // 2), :], fsem.at[i])

    for i in range(NBA):
      rdesc(i, i).start()

    def group(g, carry):
      for i in range(NBA):
        kk = g * NBA + i
        rdesc(kk, i).wait()

        @pl.when(g > 0)
        def _():
          wdesc(kk - NBA, i).wait()

        @plsc.parallel_loop(0, VBLK, step=16, unroll=4)
        def _(vv0):
          for d in range(16):
            v_idx = vv0 + jnp.bitwise_and(_iota16() + d, 15)
            r_idx = lax.shift_right_logical(v_idx, 1)
            c_par = jnp.bitwise_and(v_idx, 1) * EMBED + _iota16()
            for e0 in range(0, EMBED, 16):
              vals = plsc.load_gather(n_v.at[i], [_iota16() + e0, v_idx])
              plsc.store_scatter(f_v.at[i], [r_idx, c_par + e0], vals)
        wdesc(kk, i).start()

        @pl.when(g < GA - 1)
        def _():
          rdesc(kk + NBA, i).start()
      return carry

    lax.fori_loop(0, GA, group, 0)
    for i in range(NBA):
      wdesc((GA - 1) * NBA + i, i).wait()

  return k


def _make_gather_kernel():
  mesh = plsc.VectorSubcoreMesh(
      core_axis_name="c", subcore_axis_name="s", num_cores=NC, num_subcores=NS
  )

  @functools.partial(
      pl.kernel,
      mesh=mesh,
      compiler_params=pltpu.CompilerParams(use_tc_tiling_on_sc=True, needs_layout_passes=False),
      out_type=jax.ShapeDtypeStruct((L, EMBED, B), jnp.float32),
      scratch_types=[
          pltpu.VMEM((KB * CHUNK,), jnp.int32),
          pltpu.VMEM((KB * CHUNK,), jnp.int32),
          pltpu.VMEM((NBG, CHUNK, 2 * EMBED), jnp.float32),
          pltpu.VMEM((NBG, EMBED, CHUNK), jnp.float32),
          pltpu.SemaphoreType.DMA,
          pltpu.SemaphoreType.DMA((NBG,)),
          pltpu.SemaphoreType.DMA((NBG,)),
      ],
  )
  def k(table_hbm, idx_hbm, out_hbm, idx_v, idx2_v, g_v, s_v, isem, gsem,
        ssem):
    wid = lax.axis_index("s") * NC + lax.axis_index("c")
    pltpu.async_copy(
        idx_hbm.at[pl.ds(wid * KB * CHUNK, KB * CHUNK)], idx_v, isem).wait()

    @plsc.parallel_loop(0, KB * CHUNK, step=16, unroll=4)
    def _(o):
      idx2_v[pl.ds(o, 16)] = lax.shift_right_logical(idx_v[pl.ds(o, 16)], 1)

    def jloc(kk):
      return jnp.minimum(kk, KB - 1)

    def gdesc(kk, i):
      return pltpu.make_async_copy(
          table_hbm.at[idx2_v.at[pl.ds(jloc(kk) * CHUNK, CHUNK)]],
          g_v.at[i], gsem.at[i])

    def sdesc(kk, i):
      c = wid * KB + jloc(kk)
      ll = c // (B // CHUNK)
      b0 = pl.multiple_of((c % (B // CHUNK)) * CHUNK, CHUNK)
      return pltpu.make_async_copy(
          s_v.at[i], out_hbm.at[ll, :, pl.ds(b0, CHUNK)], ssem.at[i])

    for i in range(NBG):
      gdesc(i, i).start()

    def group(g, carry):
      for i in range(NBG):
        kk = g * NBG + i
        gdesc(kk, i).wait()

        @pl.when(g > 0)
        def _():
          sdesc(kk - NBG, i).wait()

        base = jloc(kk) * CHUNK

        @plsc.parallel_loop(0, CHUNK, step=16, unroll=4)
        def _(t0):
          t_idx = _iota16() + t0
          par = jnp.bitwise_and(idx_v[pl.ds(base + t0, 16)], 1) * EMBED
          for d in range(16):
            perm = jnp.bitwise_and(_iota16() + d, 15)
            gcol = par + perm
            for e0 in range(0, EMBED, 16):
              vals = plsc.load_gather(g_v.at[i], [t_idx, gcol + e0])
              plsc.store_scatter(s_v.at[i], [perm + e0, t_idx], vals)
        sdesc(kk, i).start()

        @pl.when(g < GB - 1)
        def _():
          gdesc(kk + NBG, i).start()
      return carry

    lax.fori_loop(0, GB, group, 0)
    for i in range(NBG):
      sdesc((GB - 1) * NBG + i, i).wait()

  return k


_format_call = _make_format_kernel()
_gather_call = _make_gather_kernel()


def kernel(token_ids, embeddings):
  table_t = embeddings.T
  idx_flat = token_ids.T.reshape(TOTAL).astype(jnp.int32)
  table2 = _format_call(table_t)
  out_t = _gather_call(table2, idx_flat)
  return out_t.transpose(2, 0, 1)
